# baseline (speedup 1.0000x reference)
import functools

import jax
import jax.numpy as jnp
from jax import lax
from jax.experimental import pallas as pl
from jax.experimental.pallas import tpu as pltpu
from jax.experimental.pallas import tpu_sc as plsc

V, D, B, L = 1000, 64, 4096, 200
VP = 1024
NC, NS = 2, 16
NW = NC * NS
HROWS = B // NW
RC = 32
NCHUNK = HROWS // RC
BB = 2048
BM = 2048
BN = 2048
NVEC = L // 16
TAIL = L - NVEC * 16


def _hist_kernel(text_hbm, cand_hbm, counts_hbm,
                 idx_v, hist0, hist1, sem0, sem1):
    wid = lax.axis_index("s") * NC + lax.axis_index("c")
    lane = lax.iota(jnp.int32, 16)
    ones = jnp.ones((16,), jnp.float32)
    zeros = jnp.zeros((16,), jnp.float32)
    tail_mask = lane >= 16 - TAIL
    hists = [hist0, hist1]
    sems = [sem0, sem1]

    def do_chunk(hist_v, sem, chunk, out_ref):
        def zero_body(r, c):
            for u in range(VP // 16):
                hist_v[r, pl.ds(u * 16, 16)] = zeros
            return c

        def scatter_body(r, c):
            rsplat = jnp.full((16,), 0, jnp.int32) + r
            ridx = chunk * RC + r
            for j in range(NVEC):
                iv = idx_v[ridx, pl.ds(j * 16, 16)]
                plsc.addupdate_scatter(hist_v, [rsplat, iv], ones)
            iv = idx_v[ridx, pl.ds(L - 16, 16)]
            plsc.addupdate_scatter(hist_v, [rsplat, iv], ones,
                                   mask=tail_mask)
            return c

        @plsc.parallel_loop(0, RC, unroll=2)
        def _(r):
            zero_body(r, 0)

        @plsc.parallel_loop(0, RC, unroll=2)
        def _(r):
            scatter_body(r, 0)

        return pltpu.async_copy(hist_v, out_ref, sem)

    pending = []
    for half, src_hbm in enumerate([text_hbm, cand_hbm]):
        pltpu.sync_copy(src_hbm.at[pl.ds(wid * HROWS, HROWS)], idx_v)
        for chunk in range(NCHUNK):
            b = (half * NCHUNK + chunk) % 2
            if len(pending) >= 2:
                pending.pop(0).wait()
            row_base = half * B + wid * HROWS + chunk * RC
            pending.append(do_chunk(
                hists[b], sems[b], chunk,
                counts_hbm.at[pl.ds(row_base, RC)]))
    for d in pending:
        d.wait()


NENC = 2 * B // BB
NSC = B // BM


def _tc_kernel(counts_ref, wc_ref, wd_ref, out_ref, e_scratch):
    s = pl.program_id(0)

    def encode(w_ref):
        w = jnp.concatenate(
            [w_ref[...], jnp.zeros((VP - V, D), jnp.float32)], axis=0)
        e = jnp.dot(
            counts_ref[...].astype(jnp.bfloat16), w.astype(jnp.bfloat16),
            preferred_element_type=jnp.float32) * (1.0 / L)
        e_scratch[pl.ds(s * BB, BB), :] = e.astype(jnp.bfloat16)

    @pl.when(s < NENC // 2)
    def _():
        encode(wc_ref)

    @pl.when((s >= NENC // 2) & (s < NENC))
    def _():
        encode(wd_ref)

    @pl.when(s >= NENC)
    def _():
        t = s - NENC
        i = t // NSC
        j = t % NSC
        a = e_scratch[pl.ds(i * BM, BM), :]
        b = e_scratch[pl.ds(B + j * BN, BN), :]
        out_ref[...] = lax.dot_general(
            a, b, (((1,), (1,)), ((), ())),
            preferred_element_type=jnp.float32)


@jax.jit
def kernel(text_vec, cand_vecs, W_ctx, W_cand):
    text_vec = text_vec.astype(jnp.int32)
    cand_vecs = cand_vecs.astype(jnp.int32)

    hist_fn = pl.kernel(
        _hist_kernel,
        out_type=jax.ShapeDtypeStruct((2 * B, VP), jnp.float32),
        mesh=plsc.VectorSubcoreMesh(
            core_axis_name="c", subcore_axis_name="s",
            num_cores=NC, num_subcores=NS),
        compiler_params=pltpu.CompilerParams(needs_layout_passes=False),
        scratch_types=[
            pltpu.VMEM((HROWS, L), jnp.int32),
            pltpu.VMEM((RC, VP), jnp.float32),
            pltpu.VMEM((RC, VP), jnp.float32),
            pltpu.SemaphoreType.DMA,
            pltpu.SemaphoreType.DMA,
        ],
    )
    counts = hist_fn(text_vec, cand_vecs)

    out = pl.pallas_call(
        _tc_kernel,
        grid=(NENC + NSC * NSC,),
        in_specs=[
            pl.BlockSpec((BB, VP), lambda s: (jnp.minimum(s, NENC - 1), 0)),
            pl.BlockSpec((V, D), lambda s: (0, 0)),
            pl.BlockSpec((V, D), lambda s: (0, 0)),
        ],
        out_specs=pl.BlockSpec(
            (BM, BN),
            lambda s: (jnp.maximum(s - NENC, 0) // NSC,
                       jnp.maximum(s - NENC, 0) % NSC)),
        out_shape=jax.ShapeDtypeStruct((B, B), jnp.float32),
        scratch_shapes=[pltpu.VMEM((2 * B, D), jnp.bfloat16)],
    )(counts, W_ctx, W_cand)
    return out


# speedup vs baseline: 1.0118x; 1.0118x over previous
import functools

import jax
import jax.numpy as jnp
from jax import lax
from jax.experimental import pallas as pl
from jax.experimental.pallas import tpu as pltpu
from jax.experimental.pallas import tpu_sc as plsc

V, D, B, L = 1000, 64, 4096, 200
VP = 1024
NC, NS = 2, 16
NW = NC * NS
HROWS = B // NW
RC = 32
NCHUNK = HROWS // RC
BB = 2048
BM = 1024
BN = 1024
NVEC = L // 16
TAIL = L - NVEC * 16


def _hist_kernel(text_hbm, cand_hbm, counts_hbm,
                 idx_v, hist0, hist1, sem0, sem1):
    wid = lax.axis_index("s") * NC + lax.axis_index("c")
    lane = lax.iota(jnp.int32, 16)
    ones = jnp.ones((16,), jnp.float32)
    zeros = jnp.zeros((16,), jnp.float32)
    tail_mask = lane >= 16 - TAIL
    hists = [hist0, hist1]
    sems = [sem0, sem1]

    def do_chunk(hist_v, sem, chunk, out_ref):
        def zero_body(r, c):
            for u in range(VP // 16):
                hist_v[r, pl.ds(u * 16, 16)] = zeros
            return c

        def scatter_body(r, c):
            rsplat = jnp.full((16,), 0, jnp.int32) + r
            ridx = chunk * RC + r
            for j in range(NVEC):
                iv = idx_v[ridx, pl.ds(j * 16, 16)]
                plsc.addupdate_scatter(hist_v, [rsplat, iv], ones)
            iv = idx_v[ridx, pl.ds(L - 16, 16)]
            plsc.addupdate_scatter(hist_v, [rsplat, iv], ones,
                                   mask=tail_mask)
            return c

        @plsc.parallel_loop(0, RC, unroll=2)
        def _(r):
            zero_body(r, 0)

        @plsc.parallel_loop(0, RC, unroll=2)
        def _(r):
            scatter_body(r, 0)

        return pltpu.async_copy(hist_v, out_ref, sem)

    pending = []
    for half, src_hbm in enumerate([text_hbm, cand_hbm]):
        pltpu.sync_copy(src_hbm.at[pl.ds(wid * HROWS, HROWS)], idx_v)
        for chunk in range(NCHUNK):
            b = (half * NCHUNK + chunk) % 2
            if len(pending) >= 2:
                pending.pop(0).wait()
            row_base = half * B + wid * HROWS + chunk * RC
            pending.append(do_chunk(
                hists[b], sems[b], chunk,
                counts_hbm.at[pl.ds(row_base, RC)]))
    for d in pending:
        d.wait()


NENC = 2 * B // BB
NSC = B // BM


def _tc_kernel(counts_ref, wc_ref, wd_ref, out_ref, e_scratch):
    s = pl.program_id(0)

    def encode(w_ref):
        w = jnp.concatenate(
            [w_ref[...], jnp.zeros((VP - V, D), jnp.float32)], axis=0)
        e = jnp.dot(
            counts_ref[...].astype(jnp.bfloat16), w.astype(jnp.bfloat16),
            preferred_element_type=jnp.float32) * (1.0 / L)
        e_scratch[pl.ds(s * BB, BB), :] = e.astype(jnp.bfloat16)

    @pl.when(s < NENC // 2)
    def _():
        encode(wc_ref)

    @pl.when((s >= NENC // 2) & (s < NENC))
    def _():
        encode(wd_ref)

    @pl.when(s >= NENC)
    def _():
        t = s - NENC
        i = t // NSC
        j = t % NSC
        a = e_scratch[pl.ds(i * BM, BM), :]
        b = e_scratch[pl.ds(B + j * BN, BN), :]
        out_ref[...] = lax.dot_general(
            a, b, (((1,), (1,)), ((), ())),
            preferred_element_type=jnp.float32)


@jax.jit
def kernel(text_vec, cand_vecs, W_ctx, W_cand):
    text_vec = text_vec.astype(jnp.int32)
    cand_vecs = cand_vecs.astype(jnp.int32)

    hist_fn = pl.kernel(
        _hist_kernel,
        out_type=jax.ShapeDtypeStruct((2 * B, VP), jnp.float32),
        mesh=plsc.VectorSubcoreMesh(
            core_axis_name="c", subcore_axis_name="s",
            num_cores=NC, num_subcores=NS),
        compiler_params=pltpu.CompilerParams(needs_layout_passes=False),
        scratch_types=[
            pltpu.VMEM((HROWS, L), jnp.int32),
            pltpu.VMEM((RC, VP), jnp.float32),
            pltpu.VMEM((RC, VP), jnp.float32),
            pltpu.SemaphoreType.DMA,
            pltpu.SemaphoreType.DMA,
        ],
    )
    counts = hist_fn(text_vec, cand_vecs)

    out = pl.pallas_call(
        _tc_kernel,
        grid=(NENC + NSC * NSC,),
        in_specs=[
            pl.BlockSpec((BB, VP), lambda s: (jnp.minimum(s, NENC - 1), 0)),
            pl.BlockSpec((V, D), lambda s: (0, 0)),
            pl.BlockSpec((V, D), lambda s: (0, 0)),
        ],
        out_specs=pl.BlockSpec(
            (BM, BN),
            lambda s: (jnp.maximum(s - NENC, 0) // NSC,
                       jnp.maximum(s - NENC, 0) % NSC)),
        out_shape=jax.ShapeDtypeStruct((B, B), jnp.float32),
        scratch_shapes=[pltpu.VMEM((2 * B, D), jnp.bfloat16)],
    )(counts, W_ctx, W_cand)
    return out
